# baseline (speedup 1.0000x reference)
import jax
import jax.numpy as jnp
from jax import lax
from jax.experimental import pallas as pl
from jax.experimental.pallas import tpu as pltpu
from jax.experimental.pallas import tpu_sc as plsc

N_FIELDS = 26
VOCAB = 100000
EMB_DIM = 32
BATCH = 16384

NC = 2
NS = 16
NW = NC * NS
TOTAL = BATCH * N_FIELDS
PER_W = TOTAL // NW

ROW_W = 128
FLAT_ROWS = N_FIELDS * VOCAB * EMB_DIM // ROW_W
VB = 128
NTC = VOCAB // VB
TAIL = VOCAB - NTC * VB
NBLK = N_FIELDS * NTC
NIT = 636
ORPB = VB * EMB_DIM // ROW_W
BLK_EL = VB * EMB_DIM
FPB = VOCAB // 4

CHUNK = 256
NCH = PER_W // CHUNK
K = 2
NGRP = NCH // K


def _transpose_body(tab_hbm, tails_hbm, out_hbm, in0, in1, in2, in3,
                    st0, st1, isem0, isem1, isem2, isem3, osem0, osem1):
    wid = lax.axis_index("s") * NC + lax.axis_index("c")
    iota = lax.iota(jnp.int32, 16)
    obase = [(vb * 4 + (iota >> 2)) * ROW_W + (iota & 3) * EMB_DIM
             for vb in range(8)]

    def bid_of(i):
        return wid + NW * i

    def active(i):
        return bid_of(i) < NBLK

    def fire_in(i, in_b, sem):
        b = bid_of(i)
        f = b // NTC
        tc = b - f * NTC
        pltpu.async_copy(tab_hbm.at[f, :, pl.ds(tc * VB, VB)], in_b, sem)

    def wait_in(in_b, sem):
        pltpu.make_async_copy(tab_hbm.at[0, :, pl.ds(0, VB)], in_b, sem).wait()

    def fire_out(i, st_b, sem):
        b = bid_of(i)
        f = b // NTC
        tc = b - f * NTC
        pltpu.async_copy(
            st_b, out_hbm.at[pl.ds((f * FPB + tc * ORPB) * ROW_W, BLK_EL)],
            sem)

    def wait_out(st_b, sem):
        pltpu.make_async_copy(st_b, out_hbm.at[pl.ds(0, BLK_EL)], sem).wait()

    def transpose_block(in_b, st_b):
        for d in range(EMB_DIM):
            vals = [in_b[d, pl.ds(vb * 16, 16)] for vb in range(8)]
            for vb in range(8):
                plsc.store_scatter(st_b, [obase[vb] + d], vals[vb])

    ins = [in0, in1, in2, in3]
    isems = [isem0, isem1, isem2, isem3]
    sts = [st0, st1]
    osems = [osem0, osem1]

    for k in range(4):
        @pl.when(active(k))
        def _(k=k):
            fire_in(k, ins[k], isems[k])

    def quad(j, carry):
        for k in range(4):
            i = 4 * j + k

            @pl.when(active(i))
            def _(i=i, k=k):
                wait_in(ins[k], isems[k])
                if k >= 2:
                    wait_out(sts[k % 2], osems[k % 2])
                else:
                    @pl.when(j > 0)
                    def _():
                        wait_out(sts[k % 2], osems[k % 2])
                transpose_block(ins[k], sts[k % 2])
                fire_out(i, sts[k % 2], osems[k % 2])

                @pl.when(active(i + 4))
                def _():
                    fire_in(i + 4, ins[k], isems[k])
        return carry

    lax.fori_loop(0, NIT // 4, quad, 0)
    wait_out(st0, osem0)
    wait_out(st1, osem1)

    @pl.when(wid < N_FIELDS)
    def _():
        pltpu.sync_copy(tails_hbm.at[wid], st0.at[pl.ds(0, 8 * ROW_W)])
        pltpu.sync_copy(
            st0.at[pl.ds(0, 8 * ROW_W)],
            out_hbm.at[pl.ds((wid * FPB + NTC * ORPB) * ROW_W, 8 * ROW_W)])


def _gather_body(idx_hbm, tab_hbm, out_hbm, idx_v, rows_v,
                 gsem0, gsem1, wsem0, wsem1):
    wid = lax.axis_index("s") * NC + lax.axis_index("c")
    base_chunk = wid * NCH

    def fire_gathers(grp, set_, sem):
        for b in range(K):
            pltpu.async_copy(tab_hbm.at[idx_v.at[grp * K + b]],
                             rows_v.at[set_, b], sem)

    def wait_gathers(set_, sem):
        for b in range(K):
            pltpu.make_async_copy(tab_hbm.at[idx_v.at[0]],
                                  rows_v.at[set_, b], sem).wait()

    def fire_wouts(grp, set_, sem):
        for b in range(K):
            c = base_chunk + grp * K + b
            pltpu.async_copy(rows_v.at[set_, b],
                             out_hbm.at[pl.ds(c * CHUNK, CHUNK)], sem)

    def wait_wouts(set_, sem):
        for b in range(K):
            pltpu.make_async_copy(rows_v.at[set_, b],
                                  out_hbm.at[pl.ds(0, CHUNK)], sem).wait()

    pltpu.sync_copy(idx_hbm.at[pl.ds(wid * NCH, NCH)], idx_v)
    fire_gathers(0, 0, gsem0)

    def pair(j, carry):
        g0 = 2 * j
        g1 = 2 * j + 1

        @pl.when(j > 0)
        def _():
            wait_wouts(1, wsem1)
        fire_gathers(g1, 1, gsem1)
        wait_gathers(0, gsem0)
        fire_wouts(g0, 0, wsem0)
        wait_wouts(0, wsem0)

        @pl.when(g0 + 2 < NGRP)
        def _():
            fire_gathers(g0 + 2, 0, gsem0)
        wait_gathers(1, gsem1)
        fire_wouts(g1, 1, wsem1)
        return carry

    lax.fori_loop(0, NGRP // 2, pair, 0)
    wait_wouts(1, wsem1)


def kernel(indices, tables):
    tab_native = jnp.transpose(tables, (0, 2, 1))
    tails = tables[:, NTC * VB:, :].reshape(N_FIELDS, 8 * ROW_W)
    mesh = plsc.VectorSubcoreMesh(core_axis_name="c", subcore_axis_name="s")

    t128 = pl.kernel(
        _transpose_body,
        mesh=mesh,
        out_type=jax.ShapeDtypeStruct((FLAT_ROWS * ROW_W,), jnp.float32),
        scratch_types=[
            pltpu.VMEM((EMB_DIM, VB), jnp.float32),
            pltpu.VMEM((EMB_DIM, VB), jnp.float32),
            pltpu.VMEM((EMB_DIM, VB), jnp.float32),
            pltpu.VMEM((EMB_DIM, VB), jnp.float32),
            pltpu.VMEM((BLK_EL,), jnp.float32),
            pltpu.VMEM((BLK_EL,), jnp.float32),
            pltpu.SemaphoreType.DMA,
            pltpu.SemaphoreType.DMA,
            pltpu.SemaphoreType.DMA,
            pltpu.SemaphoreType.DMA,
            pltpu.SemaphoreType.DMA,
            pltpu.SemaphoreType.DMA,
        ],
        compiler_params=pltpu.CompilerParams(use_tc_tiling_on_sc=True,
                                             needs_layout_passes=False),
    )(tab_native, tails)

    flat_tables = t128.reshape(N_FIELDS * VOCAB, EMB_DIM)
    offsets = jnp.arange(N_FIELDS, dtype=jnp.int32) * VOCAB
    flat_idx = (indices.astype(jnp.int32) + offsets[None, :]).reshape(
        TOTAL // CHUNK, CHUNK)

    out = pl.kernel(
        _gather_body,
        mesh=mesh,
        out_type=jax.ShapeDtypeStruct((TOTAL, EMB_DIM), jnp.float32),
        scratch_types=[
            pltpu.VMEM((NCH, CHUNK), jnp.int32),
            pltpu.VMEM((2, K, CHUNK, EMB_DIM), jnp.float32),
            pltpu.SemaphoreType.DMA,
            pltpu.SemaphoreType.DMA,
            pltpu.SemaphoreType.DMA,
            pltpu.SemaphoreType.DMA,
        ],
        compiler_params=pltpu.CompilerParams(use_tc_tiling_on_sc=False),
    )(flat_idx, flat_tables)
    return out.reshape(BATCH, N_FIELDS * EMB_DIM)


# speedup vs baseline: 1.0174x; 1.0174x over previous
import jax
import jax.numpy as jnp
from jax import lax
from jax.experimental import pallas as pl
from jax.experimental.pallas import tpu as pltpu
from jax.experimental.pallas import tpu_sc as plsc

N_FIELDS = 26
VOCAB = 100000
EMB_DIM = 32
BATCH = 16384

NC = 2
NS = 16
NW = NC * NS
TOTAL = BATCH * N_FIELDS
PER_W = TOTAL // NW

ROW_W = 128
FLAT_ROWS = N_FIELDS * VOCAB * EMB_DIM // ROW_W
VB = 128
NTC = VOCAB // VB
TAIL = VOCAB - NTC * VB
NBLK = N_FIELDS * NTC
NIT = 636
ORPB = VB * EMB_DIM // ROW_W
BLK_EL = VB * EMB_DIM
FPB = VOCAB // 4

CHUNK = 256
NCH = PER_W // CHUNK
K = 2
NGRP = NCH // K


def _transpose_body(tab_hbm, tails_hbm, out_hbm, in0, in1, in2, in3,
                    st0, st1, isem0, isem1, isem2, isem3, osem0, osem1):
    wid = lax.axis_index("s") * NC + lax.axis_index("c")
    iota = lax.iota(jnp.int32, 16)
    obase0 = (iota >> 2) * ROW_W + (iota & 3) * EMB_DIM

    def bid_of(i):
        return wid + NW * i

    def active(i):
        return bid_of(i) < NBLK

    def fire_in(i, in_b, sem):
        b = bid_of(i)
        f = b // NTC
        tc = b - f * NTC
        pltpu.async_copy(tab_hbm.at[f, :, pl.ds(tc * VB, VB)], in_b, sem)

    def wait_in(in_b, sem):
        pltpu.make_async_copy(tab_hbm.at[0, :, pl.ds(0, VB)], in_b, sem).wait()

    def fire_out(i, st_b, sem):
        b = bid_of(i)
        f = b // NTC
        tc = b - f * NTC
        pltpu.async_copy(
            st_b, out_hbm.at[pl.ds((f * FPB + tc * ORPB) * ROW_W, BLK_EL)],
            sem)

    def wait_out(st_b, sem):
        pltpu.make_async_copy(st_b, out_hbm.at[pl.ds(0, BLK_EL)], sem).wait()

    def transpose_block(in_b, st_b):
        def dstep(d, bvec):
            vals = [in_b[d, pl.ds(vb * 16, 16)] for vb in range(8)]
            for vb in range(8):
                plsc.store_scatter(st_b.at[pl.ds(vb * 512, 512)],
                                   [bvec], vals[vb])
            return bvec + 1

        lax.fori_loop(0, EMB_DIM, dstep, obase0)

    ins = [in0, in1, in2, in3]
    isems = [isem0, isem1, isem2, isem3]
    sts = [st0, st1]
    osems = [osem0, osem1]

    for k in range(4):
        @pl.when(active(k))
        def _(k=k):
            fire_in(k, ins[k], isems[k])

    def quad(j, carry):
        for k in range(4):
            i = 4 * j + k

            @pl.when(active(i))
            def _(i=i, k=k):
                wait_in(ins[k], isems[k])
                if k >= 2:
                    wait_out(sts[k % 2], osems[k % 2])
                else:
                    @pl.when(j > 0)
                    def _():
                        wait_out(sts[k % 2], osems[k % 2])
                transpose_block(ins[k], sts[k % 2])
                fire_out(i, sts[k % 2], osems[k % 2])

                @pl.when(active(i + 4))
                def _():
                    fire_in(i + 4, ins[k], isems[k])
        return carry

    lax.fori_loop(0, NIT // 4, quad, 0)
    wait_out(st0, osem0)
    wait_out(st1, osem1)

    @pl.when(wid < N_FIELDS)
    def _():
        pltpu.sync_copy(tails_hbm.at[wid], st0.at[pl.ds(0, 8 * ROW_W)])
        pltpu.sync_copy(
            st0.at[pl.ds(0, 8 * ROW_W)],
            out_hbm.at[pl.ds((wid * FPB + NTC * ORPB) * ROW_W, 8 * ROW_W)])


def _gather_body(idx_hbm, tab_hbm, out_hbm, idx_v, rows_v,
                 gsem0, gsem1, wsem0, wsem1):
    wid = lax.axis_index("s") * NC + lax.axis_index("c")
    base_chunk = wid * NCH

    def fire_gathers(grp, set_, sem):
        for b in range(K):
            pltpu.async_copy(tab_hbm.at[idx_v.at[grp * K + b]],
                             rows_v.at[set_, b], sem)

    def wait_gathers(set_, sem):
        for b in range(K):
            pltpu.make_async_copy(tab_hbm.at[idx_v.at[0]],
                                  rows_v.at[set_, b], sem).wait()

    def fire_wouts(grp, set_, sem):
        for b in range(K):
            c = base_chunk + grp * K + b
            pltpu.async_copy(rows_v.at[set_, b],
                             out_hbm.at[pl.ds(c * CHUNK, CHUNK)], sem)

    def wait_wouts(set_, sem):
        for b in range(K):
            pltpu.make_async_copy(rows_v.at[set_, b],
                                  out_hbm.at[pl.ds(0, CHUNK)], sem).wait()

    pltpu.sync_copy(idx_hbm.at[pl.ds(wid * NCH, NCH)], idx_v)
    fire_gathers(0, 0, gsem0)

    def pair(j, carry):
        g0 = 2 * j
        g1 = 2 * j + 1

        @pl.when(j > 0)
        def _():
            wait_wouts(1, wsem1)
        fire_gathers(g1, 1, gsem1)
        wait_gathers(0, gsem0)
        fire_wouts(g0, 0, wsem0)
        wait_wouts(0, wsem0)

        @pl.when(g0 + 2 < NGRP)
        def _():
            fire_gathers(g0 + 2, 0, gsem0)
        wait_gathers(1, gsem1)
        fire_wouts(g1, 1, wsem1)
        return carry

    lax.fori_loop(0, NGRP // 2, pair, 0)
    wait_wouts(1, wsem1)


def kernel(indices, tables):
    tab_native = jnp.transpose(tables, (0, 2, 1))
    tails = tables[:, NTC * VB:, :].reshape(N_FIELDS, 8 * ROW_W)
    mesh = plsc.VectorSubcoreMesh(core_axis_name="c", subcore_axis_name="s")

    t128 = pl.kernel(
        _transpose_body,
        mesh=mesh,
        out_type=jax.ShapeDtypeStruct((FLAT_ROWS * ROW_W,), jnp.float32),
        scratch_types=[
            pltpu.VMEM((EMB_DIM, VB), jnp.float32),
            pltpu.VMEM((EMB_DIM, VB), jnp.float32),
            pltpu.VMEM((EMB_DIM, VB), jnp.float32),
            pltpu.VMEM((EMB_DIM, VB), jnp.float32),
            pltpu.VMEM((BLK_EL,), jnp.float32),
            pltpu.VMEM((BLK_EL,), jnp.float32),
            pltpu.SemaphoreType.DMA,
            pltpu.SemaphoreType.DMA,
            pltpu.SemaphoreType.DMA,
            pltpu.SemaphoreType.DMA,
            pltpu.SemaphoreType.DMA,
            pltpu.SemaphoreType.DMA,
        ],
        compiler_params=pltpu.CompilerParams(use_tc_tiling_on_sc=True,
                                             needs_layout_passes=False),
    )(tab_native, tails)

    flat_tables = t128.reshape(N_FIELDS * VOCAB, EMB_DIM)
    offsets = jnp.arange(N_FIELDS, dtype=jnp.int32) * VOCAB
    flat_idx = (indices.astype(jnp.int32) + offsets[None, :]).reshape(
        TOTAL // CHUNK, CHUNK)

    out = pl.kernel(
        _gather_body,
        mesh=mesh,
        out_type=jax.ShapeDtypeStruct((TOTAL, EMB_DIM), jnp.float32),
        scratch_types=[
            pltpu.VMEM((NCH, CHUNK), jnp.int32),
            pltpu.VMEM((2, K, CHUNK, EMB_DIM), jnp.float32),
            pltpu.SemaphoreType.DMA,
            pltpu.SemaphoreType.DMA,
            pltpu.SemaphoreType.DMA,
            pltpu.SemaphoreType.DMA,
        ],
        compiler_params=pltpu.CompilerParams(use_tc_tiling_on_sc=False),
    )(flat_idx, flat_tables)
    return out.reshape(BATCH, N_FIELDS * EMB_DIM)


# speedup vs baseline: 1.3359x; 1.3131x over previous
import jax
import jax.numpy as jnp
from jax import lax
from jax.experimental import pallas as pl
from jax.experimental.pallas import tpu as pltpu
from jax.experimental.pallas import tpu_sc as plsc

N_FIELDS = 26
VOCAB = 100000
EMB_DIM = 32
BATCH = 16384

NC = 2
NS = 16
NW = NC * NS
TOTAL = BATCH * N_FIELDS
PER_W = TOTAL // NW
CHUNK = 256
NCH = PER_W // CHUNK
K = 2
NGRP = NCH // K
NPAIR = NGRP // 2


def _gather_body(idx_hbm, tab_hbm, out_hbm, idx_v, rows_v,
                 gsem0, gsem1, wsem0, wsem1):
    wid = lax.axis_index("s") * NC + lax.axis_index("c")
    base_chunk = wid * NCH

    def fire_gathers(grp, set_, sem):
        for b in range(K):
            pltpu.async_copy(tab_hbm.at[idx_v.at[grp * K + b]],
                             rows_v.at[set_, b], sem)

    def wait_gathers(set_, sem):
        for b in range(K):
            pltpu.make_async_copy(tab_hbm.at[idx_v.at[0]],
                                  rows_v.at[set_, b], sem).wait()

    def fire_wouts(grp, set_, sem):
        for b in range(K):
            c = base_chunk + grp * K + b
            pltpu.async_copy(rows_v.at[set_, b],
                             out_hbm.at[pl.ds(c * CHUNK, CHUNK)], sem)

    def wait_wouts(set_, sem):
        for b in range(K):
            pltpu.make_async_copy(rows_v.at[set_, b],
                                  out_hbm.at[pl.ds(0, CHUNK)], sem).wait()

    pltpu.sync_copy(idx_hbm.at[pl.ds(wid * NCH, NCH)], idx_v)
    fire_gathers(0, 0, gsem0)

    def pair(j, carry):
        g0 = 2 * j
        g1 = 2 * j + 1

        @pl.when(j > 0)
        def _():
            wait_wouts(1, wsem1)
        fire_gathers(g1, 1, gsem1)
        wait_gathers(0, gsem0)
        fire_wouts(g0, 0, wsem0)
        wait_wouts(0, wsem0)

        @pl.when(g0 + 2 < NGRP)
        def _():
            fire_gathers(g0 + 2, 0, gsem0)
        wait_gathers(1, gsem1)
        fire_wouts(g1, 1, wsem1)
        return carry

    lax.fori_loop(0, NPAIR, pair, 0)
    wait_wouts(1, wsem1)


def kernel(indices, tables):
    flat_tables = tables.reshape(N_FIELDS * VOCAB, EMB_DIM)
    offsets = jnp.arange(N_FIELDS, dtype=jnp.int32) * VOCAB
    flat_idx = (indices.astype(jnp.int32) + offsets[None, :]).reshape(
        TOTAL // CHUNK, CHUNK)

    mesh = plsc.VectorSubcoreMesh(core_axis_name="c", subcore_axis_name="s")
    out = pl.kernel(
        _gather_body,
        mesh=mesh,
        out_type=jax.ShapeDtypeStruct((TOTAL, EMB_DIM), jnp.float32),
        scratch_types=[
            pltpu.VMEM((NCH, CHUNK), jnp.int32),
            pltpu.VMEM((2, K, CHUNK, EMB_DIM), jnp.float32),
            pltpu.SemaphoreType.DMA,
            pltpu.SemaphoreType.DMA,
            pltpu.SemaphoreType.DMA,
            pltpu.SemaphoreType.DMA,
        ],
        compiler_params=pltpu.CompilerParams(use_tc_tiling_on_sc=False),
    )(flat_idx, flat_tables)
    return out.reshape(BATCH, N_FIELDS * EMB_DIM)
